# baseline (speedup 1.0000x reference)
import functools

import jax
import jax.numpy as jnp
from jax import lax
from jax.experimental import pallas as pl
from jax.experimental.pallas import tpu as pltpu
from jax.experimental.pallas import tpu_sc as plsc

_VOCAB = 100000
_DIM = 64
_B = 16384
_NEG = 20

_NC = 2
_NS = 16
_NW = _NC * _NS
_CH = 512


def _sc_gather(center_table, context_table, cidx, xidx, nidx):
    mesh = plsc.VectorSubcoreMesh(core_axis_name="c", subcore_axis_name="s")
    n_total = nidx.shape[0]
    c_per_w = _B // _NW
    n_per_w = n_total // _NW

    n_chunks = n_per_w // _CH

    @functools.partial(
        pl.kernel,
        mesh=mesh,
        compiler_params=pltpu.CompilerParams(use_tc_tiling_on_sc=False),
        out_type=[
            jax.ShapeDtypeStruct((_B, _DIM), jnp.float32),
            jax.ShapeDtypeStruct((_B, _DIM), jnp.float32),
            jax.ShapeDtypeStruct((n_total, _DIM), jnp.float32),
        ],
        scratch_types=[
            pltpu.VMEM((c_per_w,), jnp.int32),
            pltpu.VMEM((c_per_w,), jnp.int32),
            pltpu.VMEM((n_per_w,), jnp.int32),
            pltpu.VMEM((_CH, _DIM), jnp.float32),
            pltpu.VMEM((_CH, _DIM), jnp.float32),
            pltpu.SemaphoreType.DMA,
            pltpu.SemaphoreType.DMA,
            pltpu.SemaphoreType.DMA,
        ],
    )
    def k(ctab_hbm, xtab_hbm, cidx_hbm, xidx_hbm, nidx_hbm,
          c_out, x_out, n_out, cidx_v, xidx_v, nidx_v,
          rows_a, rows_b, gsem, wsem_a, wsem_b):
        wid = lax.axis_index("s") * _NC + lax.axis_index("c")
        cbase = wid * c_per_w
        nbase = wid * n_per_w
        rows = (rows_a, rows_b)
        wsems = (wsem_a, wsem_b)

        pltpu.sync_copy(cidx_hbm.at[pl.ds(cbase, c_per_w)], cidx_v)
        pltpu.sync_copy(xidx_hbm.at[pl.ds(cbase, c_per_w)], xidx_v)
        pltpu.sync_copy(nidx_hbm.at[pl.ds(nbase, n_per_w)], nidx_v)

        pltpu.async_copy(ctab_hbm.at[cidx_v], rows_a, gsem).wait()
        pltpu.async_copy(rows_a, c_out.at[pl.ds(cbase, c_per_w)], wsem_a)
        pltpu.async_copy(xtab_hbm.at[xidx_v], rows_b, gsem).wait()
        pltpu.async_copy(rows_b, x_out.at[pl.ds(cbase, c_per_w)], wsem_b)

        @pl.loop(0, n_chunks // 2)
        def _(g):
            for b in range(2):
                base = (g * 2 + b) * _CH
                pltpu.make_async_copy(
                    rows[b], n_out.at[pl.ds(0, _CH)], wsems[b]).wait()
                pltpu.async_copy(
                    xtab_hbm.at[nidx_v.at[pl.ds(base, _CH)]], rows[b], gsem
                ).wait()
                pltpu.async_copy(
                    rows[b], n_out.at[pl.ds(nbase + base, _CH)], wsems[b])

        pltpu.make_async_copy(rows_a, n_out.at[pl.ds(0, _CH)], wsem_a).wait()
        pltpu.make_async_copy(rows_b, n_out.at[pl.ds(0, _CH)], wsem_b).wait()

    return k(center_table, context_table, cidx, xidx, nidx)


_BSP = 1024
_EPS = 1e-12


def _half_sums(v):
    return jnp.concatenate(
        [jnp.sum(v[:, :_DIM], axis=1, keepdims=True),
         jnp.sum(v[:, _DIM:], axis=1, keepdims=True)], axis=1)


def _score_loss(p, ss, sign):
    s = p * jax.lax.rsqrt(jnp.maximum(ss, _EPS * _EPS))
    u = s * s
    even = 0.6931471805599453 + u * (
        0.125 + u * (-0.005208333333333333 + u * 0.00034722222222222224))
    return jnp.sum(even + (0.5 * sign) * s)


def _tc_body(c_ref, x_ref, n_ref, pos_ref, neg_ref, cn_ref):
    j = pl.program_id(1)

    @pl.when(j == 0)
    def _():
        pos_ref[...] = jnp.zeros((1, 1, 1), jnp.float32)
        neg_ref[...] = jnp.zeros((1, 1, 1), jnp.float32)
        c = c_ref[...]
        cl = c[:, :_DIM]
        cr = c[:, _DIM:]
        inv_l = jax.lax.rsqrt(jnp.maximum(
            jnp.sum(cl * cl, axis=1, keepdims=True), _EPS * _EPS))
        inv_r = jax.lax.rsqrt(jnp.maximum(
            jnp.sum(cr * cr, axis=1, keepdims=True), _EPS * _EPS))
        cn_ref[:, :_DIM] = cl * inv_l
        cn_ref[:, _DIM:] = cr * inv_r
        x = x_ref[...]
        cn = cn_ref[...]
        pos_ref[...] += _score_loss(
            _half_sums(x * cn), _half_sums(x * x), -1.0).reshape(1, 1, 1)

    n = n_ref[...]
    cn = cn_ref[...]
    neg_ref[...] += _score_loss(
        _half_sums(n * cn), _half_sums(n * n), 1.0).reshape(1, 1, 1)


def _tc_loss(c_rows, x_rows, n_rows):
    c128 = c_rows.reshape(_B // 2, 2 * _DIM)
    x128 = x_rows.reshape(_B // 2, 2 * _DIM)
    n128 = n_rows.reshape(_B * _NEG // 2, 2 * _DIM)
    nb = (_B // 2) // _BSP
    pos_s, neg_s = pl.pallas_call(
        _tc_body,
        grid=(nb, _NEG),
        in_specs=[
            pl.BlockSpec((_BSP, 2 * _DIM), lambda i, j: (i, 0)),
            pl.BlockSpec((_BSP, 2 * _DIM), lambda i, j: (i, 0)),
            pl.BlockSpec((_BSP, 2 * _DIM), lambda i, j: (j * nb + i, 0)),
        ],
        out_specs=[
            pl.BlockSpec((1, 1, 1), lambda i, j: (i, 0, 0)),
            pl.BlockSpec((1, 1, 1), lambda i, j: (i, 0, 0)),
        ],
        out_shape=[
            jax.ShapeDtypeStruct((nb, 1, 1), jnp.float32),
            jax.ShapeDtypeStruct((nb, 1, 1), jnp.float32),
        ],
        scratch_shapes=[pltpu.VMEM((_BSP, 2 * _DIM), jnp.float32)],
        compiler_params=pltpu.CompilerParams(
            dimension_semantics=("parallel", "arbitrary")),
    )(c128, x128, n128)
    return jnp.sum(pos_s), jnp.sum(neg_s)


def kernel(center_table, context_table, center_word, context_word, negative_samples):
    cidx = jnp.clip(center_word, 0, _VOCAB - 1).astype(jnp.int32)
    xidx = jnp.clip(context_word, 0, _VOCAB - 1).astype(jnp.int32)
    nidx = jnp.clip(negative_samples, 0, _VOCAB - 1).astype(jnp.int32).T.reshape(-1)

    c_rows, x_rows, n_rows = _sc_gather(center_table, context_table, cidx, xidx, nidx)
    pos_sum, neg_sum = _tc_loss(c_rows, x_rows, n_rows)

    pos_loss = pos_sum / _B
    neg_loss = neg_sum / (_B * _NEG)
    return (pos_loss + neg_loss, pos_loss, neg_loss)


# speedup vs baseline: 1.6681x; 1.6681x over previous
import functools

import jax
import jax.numpy as jnp
from jax import lax
from jax.experimental import pallas as pl
from jax.experimental.pallas import tpu as pltpu
from jax.experimental.pallas import tpu_sc as plsc

_VOCAB = 100000
_DIM = 64
_B = 16384
_NEG = 20

_NC = 2
_NS = 16
_NW = _NC * _NS
_CH = 512


def _sc_gather(center_table, context_table, cidx, xidx, nidx):
    mesh = plsc.VectorSubcoreMesh(core_axis_name="c", subcore_axis_name="s")
    n_total = nidx.shape[0]
    c_per_w = _B // _NW
    n_per_w = n_total // _NW

    n_chunks = n_per_w // _CH

    @functools.partial(
        pl.kernel,
        mesh=mesh,
        compiler_params=pltpu.CompilerParams(use_tc_tiling_on_sc=False),
        out_type=[
            jax.ShapeDtypeStruct((_B, _DIM), jnp.float32),
            jax.ShapeDtypeStruct((_B, _DIM), jnp.float32),
            jax.ShapeDtypeStruct((n_total, _DIM), jnp.float32),
        ],
        scratch_types=[
            pltpu.VMEM((c_per_w,), jnp.int32),
            pltpu.VMEM((c_per_w,), jnp.int32),
            pltpu.VMEM((n_per_w,), jnp.int32),
            pltpu.VMEM((_CH, _DIM), jnp.float32),
            pltpu.VMEM((_CH, _DIM), jnp.float32),
            pltpu.SemaphoreType.DMA,
            pltpu.SemaphoreType.DMA,
            pltpu.SemaphoreType.DMA,
        ],
    )
    def k(ctab_hbm, xtab_hbm, cidx_hbm, xidx_hbm, nidx_hbm,
          c_out, x_out, n_out, cidx_v, xidx_v, nidx_v,
          rows_a, rows_b, gsem, wsem_a, wsem_b):
        wid = lax.axis_index("s") * _NC + lax.axis_index("c")
        cbase = wid * c_per_w
        nbase = wid * n_per_w
        rows = (rows_a, rows_b)
        wsems = (wsem_a, wsem_b)

        pltpu.sync_copy(cidx_hbm.at[pl.ds(cbase, c_per_w)], cidx_v)
        pltpu.sync_copy(xidx_hbm.at[pl.ds(cbase, c_per_w)], xidx_v)
        pltpu.sync_copy(nidx_hbm.at[pl.ds(nbase, n_per_w)], nidx_v)

        pltpu.async_copy(ctab_hbm.at[cidx_v], rows_a, gsem).wait()
        pltpu.async_copy(rows_a, c_out.at[pl.ds(cbase, c_per_w)], wsem_a)
        pltpu.async_copy(xtab_hbm.at[xidx_v], rows_b, gsem).wait()
        pltpu.async_copy(rows_b, x_out.at[pl.ds(cbase, c_per_w)], wsem_b)

        @pl.loop(0, n_chunks // 2)
        def _(g):
            for b in range(2):
                base = (g * 2 + b) * _CH
                pltpu.make_async_copy(
                    rows[b], n_out.at[pl.ds(0, _CH)], wsems[b]).wait()
                pltpu.async_copy(
                    xtab_hbm.at[nidx_v.at[pl.ds(base, _CH)]], rows[b], gsem
                ).wait()
                pltpu.async_copy(
                    rows[b], n_out.at[pl.ds(nbase + base, _CH)], wsems[b])

        pltpu.make_async_copy(rows_a, n_out.at[pl.ds(0, _CH)], wsem_a).wait()
        pltpu.make_async_copy(rows_b, n_out.at[pl.ds(0, _CH)], wsem_b).wait()

    return k(center_table, context_table, cidx, xidx, nidx)


_BSP = 1024
_EPS = 1e-12


def _half_selector():
    lane = lax.broadcasted_iota(jnp.int32, (2 * _DIM, 2), 0)
    col = lax.broadcasted_iota(jnp.int32, (2 * _DIM, 2), 1)
    return ((lane < _DIM) == (col == 0)).astype(jnp.float32)


def _half_dots(v, w, sel):
    return jax.lax.dot_general(
        v * w, sel, (((1,), (0,)), ((), ())),
        preferred_element_type=jnp.float32)


def _score_loss(p, ss, sign):
    s = p * jax.lax.rsqrt(jnp.maximum(ss, _EPS * _EPS))
    u = s * s
    even = 0.6931471805599453 + u * (
        0.125 + u * (-0.005208333333333333 + u * 0.00034722222222222224))
    return even + (0.5 * sign) * s


def _tc_body(c_ref, x_ref, n_ref, pos_ref, neg_ref, invc_ref, acc_ref):
    j = pl.program_id(1)
    sel = _half_selector()
    c = c_ref[...]

    @pl.when(j == 0)
    def _():
        invc = jax.lax.rsqrt(jnp.maximum(
            _half_dots(c, c, sel), _EPS * _EPS))
        invc_ref[...] = invc
        x = x_ref[...]
        pos = _score_loss(
            _half_dots(x, c, sel) * invc,
            _half_dots(x, x, sel), -1.0)
        pos_ref[...] = jnp.sum(pos).reshape(1, 1, 1)

    n = n_ref[...]
    invc = invc_ref[...]
    neg = _score_loss(
        _half_dots(n, c, sel) * invc,
        _half_dots(n, n, sel), 1.0)

    @pl.when(j == 0)
    def _():
        acc_ref[...] = jnp.zeros((_BSP, 2), jnp.float32)

    acc_ref[...] += neg

    @pl.when(j == _NEG - 1)
    def _():
        neg_ref[...] = jnp.sum(acc_ref[...]).reshape(1, 1, 1)


def _tc_loss(c_rows, x_rows, n_rows):
    c128 = c_rows.reshape(_B // 2, 2 * _DIM)
    x128 = x_rows.reshape(_B // 2, 2 * _DIM)
    n128 = n_rows.reshape(_B * _NEG // 2, 2 * _DIM)
    nb = (_B // 2) // _BSP
    pos_s, neg_s = pl.pallas_call(
        _tc_body,
        grid=(nb, _NEG),
        in_specs=[
            pl.BlockSpec((_BSP, 2 * _DIM), lambda i, j: (i, 0)),
            pl.BlockSpec((_BSP, 2 * _DIM), lambda i, j: (i, 0)),
            pl.BlockSpec((_BSP, 2 * _DIM), lambda i, j: (j * nb + i, 0)),
        ],
        out_specs=[
            pl.BlockSpec((1, 1, 1), lambda i, j: (i, 0, 0)),
            pl.BlockSpec((1, 1, 1), lambda i, j: (i, 0, 0)),
        ],
        out_shape=[
            jax.ShapeDtypeStruct((nb, 1, 1), jnp.float32),
            jax.ShapeDtypeStruct((nb, 1, 1), jnp.float32),
        ],
        scratch_shapes=[
            pltpu.VMEM((_BSP, 2), jnp.float32),
            pltpu.VMEM((_BSP, 2), jnp.float32),
        ],
        compiler_params=pltpu.CompilerParams(
            dimension_semantics=("parallel", "arbitrary")),
    )(c128, x128, n128)
    return jnp.sum(pos_s), jnp.sum(neg_s)


def kernel(center_table, context_table, center_word, context_word, negative_samples):
    cidx = jnp.clip(center_word, 0, _VOCAB - 1).astype(jnp.int32)
    xidx = jnp.clip(context_word, 0, _VOCAB - 1).astype(jnp.int32)
    nidx = jnp.clip(negative_samples, 0, _VOCAB - 1).astype(jnp.int32).T.reshape(-1)

    c_rows, x_rows, n_rows = _sc_gather(center_table, context_table, cidx, xidx, nidx)
    pos_sum, neg_sum = _tc_loss(c_rows, x_rows, n_rows)

    pos_loss = pos_sum / _B
    neg_loss = neg_sum / (_B * _NEG)
    return (pos_loss + neg_loss, pos_loss, neg_loss)
